# baseline (speedup 1.0000x reference)
import functools

import jax
import jax.numpy as jnp
from jax import lax
from jax.experimental import pallas as pl
from jax.experimental.pallas import tpu as pltpu
from jax.experimental.pallas import tpu_sc as plsc

_BS = 64
_L = 16


def _sc_body(catT_hbm, out_hbm, src_v, tile_v, sem):
    n2d, w = catT_hbm.shape
    cpw = src_v.shape[0]
    hw = tile_v.shape[1]
    groups = hw // _L
    nc = lax.axis_index("c")
    ns = lax.axis_index("s")
    wid = ns * 2 + nc
    woff = pl.multiple_of(wid * cpw, cpw)
    pltpu.sync_copy(catT_hbm.at[pl.ds(woff, cpw)], src_v)

    iota = lax.broadcasted_iota(jnp.int32, (_L,), 0)
    wv = jnp.zeros((_L,), jnp.int32) + wid
    is_top = wv < (n2d // (2 * cpw))

    def build(i, carry):
        cc = i // groups
        g = i % groups
        lane = g * _L + iota
        col_idx = jnp.where(is_top, lane % w, lane // w)
        row_idx = jnp.zeros((_L,), jnp.int32) + cc
        val = plsc.load_gather(src_v, [row_idx, col_idx])
        plsc.store_scatter(tile_v, [row_idx, lane], val)
        return carry

    lax.fori_loop(0, cpw * groups, build, 0)

    copies = [
        pltpu.make_async_copy(tile_v, out_hbm.at[b, pl.ds(woff, cpw)], sem)
        for b in range(_BS)
    ]
    for cp in copies:
        cp.start()
    for cp in copies:
        cp.wait()


def kernel(row_embed, col_embed, bs):
    h, d = row_embed.shape
    w = col_embed.shape[0]
    catT = jnp.concatenate([col_embed.T, row_embed.T], axis=0)
    cpw = (2 * d) // 32
    sck = pl.kernel(
        _sc_body,
        out_type=jax.ShapeDtypeStruct((_BS, 2 * d, h * w), jnp.float32),
        mesh=plsc.VectorSubcoreMesh(core_axis_name="c", subcore_axis_name="s"),
        scratch_types=[
            pltpu.VMEM((cpw, w), jnp.float32),
            pltpu.VMEM((cpw, h * w), jnp.float32),
            pltpu.SemaphoreType.DMA,
        ],
        compiler_params=pltpu.CompilerParams(
            use_tc_tiling_on_sc=True, needs_layout_passes=False),
    )
    out = sck(catT)
    return out.reshape(_BS, 2 * d, h, w)
